# baseline (speedup 1.0000x reference)
import functools

import jax
import jax.numpy as jnp
from jax.experimental import pallas as pl
from jax.experimental.pallas import tpu as pltpu

N_EDGES = 320000
N_GRAPHS = 64
RATIO = 0.5
EPS = 1e-12

_BLK = 3200


def _mlp_body(r_ref, c_ref, w1a_ref, w1b_ref, b1_ref, w2_ref, b2_ref, att_ref):
    h = jnp.dot(r_ref[...], w1a_ref[...], preferred_element_type=jnp.float32)
    h = h + jnp.dot(c_ref[...], w1b_ref[...], preferred_element_type=jnp.float32)
    h = jax.nn.relu(h + b1_ref[...])
    att = jnp.sum(h * w2_ref[...], axis=1) + b2_ref[0, 0]
    att_ref[...] = att.reshape(1, 1, -1)


def _mlp_att(R, C, W1, b1, W2, b2):
    W1a = W1[:128]
    W1b = W1[128:]
    b1r = b1.reshape(1, -1)
    w2r = W2.reshape(1, -1)
    b2r = b2.reshape(1, 1)
    grid = N_EDGES // _BLK
    return pl.pallas_call(
        _mlp_body,
        grid=(grid,),
        in_specs=[
            pl.BlockSpec((_BLK, 128), lambda i: (i, 0)),
            pl.BlockSpec((_BLK, 128), lambda i: (i, 0)),
            pl.BlockSpec((128, 512), lambda i: (0, 0)),
            pl.BlockSpec((128, 512), lambda i: (0, 0)),
            pl.BlockSpec((1, 512), lambda i: (0, 0)),
            pl.BlockSpec((1, 512), lambda i: (0, 0)),
            pl.BlockSpec((1, 1), lambda i: (0, 0), memory_space=pltpu.SMEM),
        ],
        out_specs=pl.BlockSpec((1, 1, _BLK), lambda i: (i, 0, 0)),
        out_shape=jax.ShapeDtypeStruct((grid, 1, _BLK), jnp.float32),
    )(R, C, W1a, W1b, b1r, w2r, b2r).reshape(N_EDGES)


def kernel(emb, edge_index, node_batch, W1, b1, W2, b2):
    row = edge_index[0]
    col = edge_index[1]
    R = jnp.take(emb, row, axis=0)
    C = jnp.take(emb, col, axis=0)
    att = _mlp_att(R, C, W1, b1, W2, b2)
    index = jnp.take(node_batch, row)

    f_min = att.min()
    f_max = att.max()
    norm = (att - f_min) / (f_max - f_min + EPS) + index.astype(jnp.float32) * (-1.0)
    perm = jnp.argsort(-norm)
    deg = jnp.bincount(index, length=N_GRAPHS)
    k = jnp.ceil(RATIO * deg.astype(jnp.float32)).astype(jnp.int32)
    cum = jnp.cumsum(deg)
    start = jnp.concatenate([jnp.zeros((1,), dtype=deg.dtype), cum])
    pos = jnp.arange(att.shape[0])
    g = jnp.searchsorted(cum, pos, side='right')
    mask = (pos - jnp.take(start, g)) < jnp.take(k, g)
    sorted_att = jnp.take(att, perm)
    signed = jnp.where(mask, sorted_att, -sorted_att)
    order = jnp.argsort(jnp.logical_not(mask), stable=True)
    return jnp.take(signed, order)
